# baseline (speedup 1.0000x reference)
import jax
import jax.numpy as jnp
from jax import lax
from jax.experimental import pallas as pl
from jax.experimental.pallas import tpu as pltpu
from jax.experimental.pallas import tpu_sc as plsc

_B = 16384
_D = 32

_info = plsc.get_sparse_core_info()
_NC, _NS = _info.num_cores, _info.num_subcores
_NW = _NC * _NS
_BPW = _B // _NW
_CH = 128
_NCH = _BPW // _CH


def _mf_bpr_body(user_h, item_i_h, item_j_h, user_t, item_t,
                 out_i_h, out_j_h,
                 idx_u, idx_i, idx_j, ru, ri, rj, oi, oj, sem):
    wid = lax.axis_index("s") * _NC + lax.axis_index("c")
    base = wid * _BPW

    pltpu.sync_copy(user_h.at[pl.ds(base, _BPW)], idx_u)
    pltpu.sync_copy(item_i_h.at[pl.ds(base, _BPW)], idx_i)
    pltpu.sync_copy(item_j_h.at[pl.ds(base, _BPW)], idx_j)

    copies = []
    for k in range(_NCH):
        s = pl.ds(k * _CH, _CH)
        copies.append(pltpu.async_copy(user_t.at[idx_u.at[s]], ru.at[s], sem))
        copies.append(pltpu.async_copy(item_t.at[idx_i.at[s]], ri.at[s], sem))
        copies.append(pltpu.async_copy(item_t.at[idx_j.at[s]], rj.at[s], sem))
    for c in copies:
        c.wait()

    lane = lax.iota(jnp.int32, 16)

    def group(g, carry):
        rows = g * 16 + lane
        acc_i = jnp.zeros((16,), jnp.float32)
        acc_j = jnp.zeros((16,), jnp.float32)
        for d0 in range(_D):
            col = (lane + d0) & (_D - 1)
            u = plsc.load_gather(ru, [rows, col])
            ai = plsc.load_gather(ri, [rows, col])
            bj = plsc.load_gather(rj, [rows, col])
            acc_i = acc_i + u * ai
            acc_j = acc_j + u * bj
        oi[pl.ds(g * 16, 16)] = acc_i
        oj[pl.ds(g * 16, 16)] = acc_j
        return carry

    lax.fori_loop(0, _BPW // 16, group, None)

    pltpu.sync_copy(oi, out_i_h.at[pl.ds(base, _BPW)])
    pltpu.sync_copy(oj, out_j_h.at[pl.ds(base, _BPW)])


_mf_bpr = pl.kernel(
    _mf_bpr_body,
    out_type=(
        jax.ShapeDtypeStruct((_B,), jnp.float32),
        jax.ShapeDtypeStruct((_B,), jnp.float32),
    ),
    mesh=plsc.VectorSubcoreMesh(core_axis_name="c", subcore_axis_name="s"),
    compiler_params=pltpu.CompilerParams(
        needs_layout_passes=False, use_tc_tiling_on_sc=False),
    scratch_types=[
        pltpu.VMEM((_BPW,), jnp.int32),
        pltpu.VMEM((_BPW,), jnp.int32),
        pltpu.VMEM((_BPW,), jnp.int32),
        pltpu.VMEM((_BPW, _D), jnp.float32),
        pltpu.VMEM((_BPW, _D), jnp.float32),
        pltpu.VMEM((_BPW, _D), jnp.float32),
        pltpu.VMEM((_BPW,), jnp.float32),
        pltpu.VMEM((_BPW,), jnp.float32),
        pltpu.SemaphoreType.DMA,
    ],
)


def kernel(user, item_i, item_j, user_table, item_table):
    return _mf_bpr(user, item_i, item_j, user_table, item_table)
